# baseline (speedup 1.0000x reference)
import functools

import jax
import jax.numpy as jnp
from jax import lax
from jax.experimental import pallas as pl
from jax.experimental.pallas import tpu as pltpu
from jax.experimental.pallas import tpu_sc as plsc

_BB = 2048
_SC_CB = 128
_SC_NW = 32
_SC_BS = _SC_CB * _SC_NW
_SC_WCHUNK = 10


def _consistency(adj, fp):
    f32 = jnp.float32
    m = (adj > 0).astype(f32)
    counts = jnp.sum(m, axis=0, keepdims=True)
    pf = (fp > 0.5).astype(f32)
    numer = lax.dot_general(m, pf, (((0,), (0,)), ((), ())),
                            preferred_element_type=f32)
    mpf = numer / jnp.maximum(counts.T, 1.0)
    consistent = (mpf <= fp + 0.3).astype(f32)
    return jnp.where(counts.T > 0, consistent, 1.0)


def _support(qos, bq, bs):
    qn = (qos - bq) * bs
    nsq = jnp.sum(qn * qn, axis=1)
    return jax.nn.sigmoid(jnp.sqrt(nsq))


def _main_body(w_ref, adj_ref, bq_ref, bs_ref, fp_ref, qos_ref, fh_ref,
               trust_ref, cons_ref, supp_ref, stab_ref):
    f32 = jnp.float32
    N = adj_ref.shape[0]
    W = fh_ref.shape[0] // N
    S = 8

    cons = _consistency(adj_ref[...], fp_ref[...])
    cons_ref[...] = cons

    w1 = w_ref[0]
    w2 = w_ref[1]
    w3 = w_ref[2]
    inv_w = f32(1.0 / W)

    for nb in range(N // S):
        lo = nb * S
        v = fh_ref[pl.ds(lo, S), :]
        a1 = v
        a2 = v * v
        for wi in range(1, W):
            v = fh_ref[pl.ds(wi * N + lo, S), :]
            a1 = a1 + v
            a2 = a2 + v * v
        mean = a1 * inv_w
        var = a2 * inv_w - mean * mean
        stab = 1.0 / (1.0 + var)

        supp = _support(qos_ref[pl.ds(lo, S)], bq_ref[...], bs_ref[...])

        cs = cons[lo:lo + S, :]
        trust_ref[pl.ds(lo, S), :] = w1 * cs + w2 * supp + w3 * stab
        supp_ref[pl.ds(lo, S), :] = supp
        stab_ref[pl.ds(lo, S), :] = stab


def _pre_body(w_ref, adj_ref, bq_ref, bs_ref, fp_ref, qos_ref,
              cons_ref, supp_ref, part_ref):
    cons = _consistency(adj_ref[...], fp_ref[...])
    supp = _support(qos_ref[...], bq_ref[...], bs_ref[...])
    cons_ref[...] = cons
    supp_ref[...] = supp
    part_ref[...] = w_ref[0] * cons + w_ref[1] * supp


def _sc_stab_body(N, W, bt, fh_hbm, stab_hbm, buf, a1, a2, sem):
    f32 = jnp.float32
    wid = lax.axis_index("s") * 2 + lax.axis_index("c")
    c0 = bt + wid * _SC_CB
    inv_w = f32(1.0 / W)
    rows = _SC_WCHUNK * N
    n_lp = _SC_CB // 16

    for chunk in range(W // _SC_WCHUNK):
        pltpu.async_copy(
            fh_hbm.at[pl.ds(chunk * rows, rows), pl.ds(c0, _SC_CB)],
            buf, sem).wait()

        def acc_body(n, _, first=(chunk == 0)):
            for lp in range(n_lp):
                sl = pl.ds(lp * 16, 16)
                v = buf[n, sl]
                if first:
                    r1 = v
                    r2 = v * v
                else:
                    r1 = a1[n, sl] + v
                    r2 = a2[n, sl] + v * v
                for wl in range(1, _SC_WCHUNK):
                    v = buf[wl * N + n, sl]
                    r1 = r1 + v
                    r2 = r2 + v * v
                a1[n, sl] = r1
                a2[n, sl] = r2
            return _

        lax.fori_loop(0, N, acc_body, None)

    def fin_body(n, _):
        for lp in range(n_lp):
            sl = pl.ds(lp * 16, 16)
            s1 = a1[n, sl]
            s2 = a2[n, sl]
            mean = s1 * inv_w
            var = s2 * inv_w - mean * mean
            a1[n, sl] = 1.0 / (1.0 + var)
        return _

    lax.fori_loop(0, N, fin_body, None)
    pltpu.async_copy(a1, stab_hbm.at[:, pl.ds(wid * _SC_CB, _SC_CB)], sem).wait()


def kernel(fault_probs, qos_observations, fault_history, adjacency_matrix,
           gamma1, gamma2, gamma3, baseline_qos, baseline_std):
    B, N, W = fault_history.shape
    Q = qos_observations.shape[-1]
    f32 = jnp.float32
    fh_lin = fault_history.transpose(2, 1, 0).reshape(W * N, B)
    qos3 = qos_observations.transpose(1, 2, 0)
    fp2 = fault_probs.transpose(1, 2, 0).reshape(N, B)

    gsum = gamma1 + gamma2 + gamma3 + 1e-8
    w1 = (gamma1 / gsum).astype(f32)
    w2 = (gamma2 / gsum).astype(f32)
    w3 = (gamma3 / gsum).astype(f32)
    w = jnp.stack([w1, w2, w3])
    bq = baseline_qos[None, :, None]
    bs = (1.0 / (baseline_std + 1e-8))[None, :, None]

    bt = B - _SC_BS

    scmesh = plsc.VectorSubcoreMesh(core_axis_name="c", subcore_axis_name="s")
    sc_stab = pl.kernel(
        functools.partial(_sc_stab_body, N, W, bt),
        mesh=scmesh,
        out_type=jax.ShapeDtypeStruct((N, _SC_BS), f32),
        scratch_types=[
            pltpu.VMEM((_SC_WCHUNK * N, _SC_CB), f32),
            pltpu.VMEM((N, _SC_CB), f32),
            pltpu.VMEM((N, _SC_CB), f32),
            pltpu.SemaphoreType.DMA,
        ],
    )
    stab_sc = sc_stab(fh_lin)

    n_pre = _SC_BS // _BB
    cons_sc, supp_sc, part_sc = pl.pallas_call(
        _pre_body,
        grid=(n_pre,),
        in_specs=[
            pl.BlockSpec(memory_space=pltpu.SMEM),
            pl.BlockSpec((N, N), lambda j: (0, 0)),
            pl.BlockSpec((1, Q, 1), lambda j: (0, 0, 0)),
            pl.BlockSpec((1, Q, 1), lambda j: (0, 0, 0)),
            pl.BlockSpec((N, _BB), lambda j: (0, j + bt // _BB)),
            pl.BlockSpec((N, Q, _BB), lambda j: (0, 0, j + bt // _BB)),
        ],
        out_specs=[pl.BlockSpec((N, _BB), lambda j: (0, j))] * 3,
        out_shape=[jax.ShapeDtypeStruct((N, _SC_BS), f32)] * 3,
        compiler_params=pltpu.CompilerParams(
            dimension_semantics=("arbitrary",),
        ),
    )(w, adjacency_matrix, bq, bs, fp2, qos3)

    trust_tc, cons_tc, supp_tc, stab_tc = pl.pallas_call(
        _main_body,
        grid=(bt // _BB,),
        in_specs=[
            pl.BlockSpec(memory_space=pltpu.SMEM),
            pl.BlockSpec((N, N), lambda j: (0, 0)),
            pl.BlockSpec((1, Q, 1), lambda j: (0, 0, 0)),
            pl.BlockSpec((1, Q, 1), lambda j: (0, 0, 0)),
            pl.BlockSpec((N, _BB), lambda j: (0, j)),
            pl.BlockSpec((N, Q, _BB), lambda j: (0, 0, j)),
            pl.BlockSpec((W * N, _BB), lambda j: (0, j)),
        ],
        out_specs=[pl.BlockSpec((N, _BB), lambda j: (0, j))] * 4,
        out_shape=[jax.ShapeDtypeStruct((N, bt), f32)] * 4,
        compiler_params=pltpu.CompilerParams(
            dimension_semantics=("arbitrary",),
        ),
    )(w, adjacency_matrix, bq, bs, fp2, qos3, fh_lin)

    trust_sc = part_sc + w3 * stab_sc

    def back(tc_piece, sc_piece):
        return jnp.concatenate([tc_piece, sc_piece], axis=1).T[:, :, None]

    return (back(trust_tc, trust_sc), back(cons_tc, cons_sc),
            back(supp_tc, supp_sc), back(stab_tc, stab_sc))


# speedup vs baseline: 1.6230x; 1.6230x over previous
import jax
import jax.numpy as jnp
from jax import lax
from jax.experimental import pallas as pl
from jax.experimental.pallas import tpu as pltpu

_BB = 2048


def _body(w_ref, adj_ref, bq_ref, bs_ref, fp_ref, qos_ref, fh_ref,
          trust_ref, cons_ref, supp_ref, stab_ref):
    f32 = jnp.float32
    N = adj_ref.shape[0]
    W = fh_ref.shape[0] // N
    S = 8

    fp = fp_ref[...]
    m = (adj_ref[...] > 0).astype(f32)
    counts = jnp.sum(m, axis=0, keepdims=True)
    pf = (fp > 0.5).astype(f32)
    numer = lax.dot_general(m, pf, (((0,), (0,)), ((), ())),
                            preferred_element_type=f32)
    mpf = numer / jnp.maximum(counts.T, 1.0)
    consistent = (mpf <= fp + 0.3).astype(f32)
    cons = jnp.where(counts.T > 0, consistent, 1.0)

    w1 = w_ref[0]
    w2 = w_ref[1]
    w3 = w_ref[2]
    inv_w = f32(1.0 / W)

    for nb in range(N // S):
        lo = nb * S
        v = fh_ref[pl.ds(lo, S), :]
        a1 = v
        a2 = v * v
        for wi in range(1, W):
            v = fh_ref[pl.ds(wi * N + lo, S), :]
            a1 = a1 + v
            a2 = a2 + v * v
        mean = a1 * inv_w
        var = a2 * inv_w - mean * mean
        stab = 1.0 / (1.0 + var)

        qn = (qos_ref[pl.ds(lo, S)] - bq_ref[...]) * bs_ref[...]
        nsq = jnp.sum(qn * qn, axis=1)
        supp = jax.nn.sigmoid(jnp.sqrt(nsq))

        cs = cons[lo:lo + S, :]
        trust = w1 * cs + w2 * supp + w3 * stab
        nt = supp.shape[1] // 128
        trust_ref[pl.ds(lo, S)] = trust.reshape(S, nt, 128)
        cons_ref[pl.ds(lo, S)] = cs.reshape(S, nt, 128)
        supp_ref[pl.ds(lo, S)] = supp.reshape(S, nt, 128)
        stab_ref[pl.ds(lo, S)] = stab.reshape(S, nt, 128)


def kernel(fault_probs, qos_observations, fault_history, adjacency_matrix,
           gamma1, gamma2, gamma3, baseline_qos, baseline_std):
    B, N, W = fault_history.shape
    Q = qos_observations.shape[-1]
    fh_lin = fault_history.transpose(2, 1, 0).reshape(W * N, B)
    qos3 = qos_observations.transpose(1, 2, 0)
    fp2 = fault_probs.transpose(1, 2, 0).reshape(N, B)

    gsum = gamma1 + gamma2 + gamma3 + 1e-8
    w = jnp.stack([gamma1 / gsum, gamma2 / gsum, gamma3 / gsum]).astype(jnp.float32)
    bq = baseline_qos[None, :, None]
    bs = (1.0 / (baseline_std + 1e-8))[None, :, None]

    grid = (B // _BB,)
    out_shape = jax.ShapeDtypeStruct((N, B // 128, 128), jnp.float32)

    trust, cons, supp, stab = pl.pallas_call(
        _body,
        grid=grid,
        in_specs=[
            pl.BlockSpec(memory_space=pltpu.SMEM),
            pl.BlockSpec((N, N), lambda j: (0, 0)),
            pl.BlockSpec((1, Q, 1), lambda j: (0, 0, 0)),
            pl.BlockSpec((1, Q, 1), lambda j: (0, 0, 0)),
            pl.BlockSpec((N, _BB), lambda j: (0, j)),
            pl.BlockSpec((N, Q, _BB), lambda j: (0, 0, j)),
            pl.BlockSpec((W * N, _BB), lambda j: (0, j)),
        ],
        out_specs=[pl.BlockSpec((N, _BB // 128, 128), lambda j: (0, j, 0))] * 4,
        out_shape=[out_shape] * 4,
        compiler_params=pltpu.CompilerParams(
            dimension_semantics=("arbitrary",),
        ),
    )(w, adjacency_matrix, bq, bs, fp2, qos3, fh_lin)

    def back(a):
        return a.transpose(1, 2, 0).reshape(B, N)[:, :, None]

    return back(trust), back(cons), back(supp), back(stab)
